# Initial kernel scaffold; baseline (speedup 1.0000x reference)
import jax
import jax.numpy as jnp
from jax.experimental import pallas as pl


def kernel(x, rope_cos, rope_sin, Wq, Wk, Wv, Wo, layer_idx):
    raise NotImplementedError("write your pallas kernel here")


# speedup vs baseline: 861.4790x; 861.4790x over previous
import math

import jax
import jax.numpy as jnp
import numpy as np
from jax.experimental import pallas as pl
from jax.experimental.pallas import tpu as pltpu

B = 2
S = 2048
DIM = 1024
H = 16
DH = 64
BLOCK = 128
NB = S // BLOCK
NCH = S // BLOCK
TOPK = 8
NEG_INF = -10000.0
SCALE = 1.0 / math.sqrt(DH)

_TS = 512


def _deinterleave_perm():
    perm = np.empty((DIM,), dtype=np.int32)
    for h in range(H):
        for j in range(DH):
            src = 2 * j if j < DH // 2 else 2 * (j - DH // 2) + 1
            perm[h * DH + j] = h * DH + src
    return perm


_PERM = _deinterleave_perm()


def _qkv_body(x_ref, w_ref, q_ref, k_ref, v_ref):
    acc = jnp.dot(x_ref[0], w_ref[...], preferred_element_type=jnp.float32)
    for h in range(H):
        q_ref[0, h, :, :] = acc[:, h * DH:(h + 1) * DH]
        k_ref[0, h, :, :] = acc[:, DIM + h * DH:DIM + (h + 1) * DH]
        v_ref[0, h, :, :] = acc[:, 2 * DIM + h * DH:2 * DIM + (h + 1) * DH]


def _qkv_proj(x3d, wqkv):
    out = jax.ShapeDtypeStruct((B, H, S, DH), jnp.float32)
    ospec = pl.BlockSpec((1, H, _TS, DH), lambda b, i: (b, 0, i, 0))
    return pl.pallas_call(
        _qkv_body,
        grid=(B, S // _TS),
        in_specs=[
            pl.BlockSpec((1, _TS, DIM), lambda b, i: (b, i, 0)),
            pl.BlockSpec((DIM, 3 * DIM), lambda b, i: (0, 0)),
        ],
        out_specs=[ospec, ospec, ospec],
        out_shape=[out, out, out],
    )(x3d, wqkv)


def _rope(t, cos64, sin_sgn):
    swapped = jnp.concatenate([t[:, DH // 2:], t[:, :DH // 2]], axis=1)
    return t * cos64 + swapped * sin_sgn


def _select_body(q_ref, k_ref, cos_ref, sin_ref, qr_ref, kr_ref, sel_ref):
    q = q_ref[0, 0, :, :]
    k = k_ref[0, 0, :, :]
    cos64 = cos_ref[...]
    sin_sgn = sin_ref[...]
    qr = _rope(q, cos64, sin_sgn)
    kr = _rope(k, cos64, sin_sgn)
    qr_ref[0, 0, :, :] = qr
    kr_ref[0, 0, :, :] = kr

    br = jnp.mean(kr.reshape(NB, BLOCK, DH), axis=1)
    q_mid = qr.reshape(NCH, BLOCK, DH)[:, BLOCK // 2, :]
    gating = jax.lax.dot_general(
        q_mid, br, (((1,), (1,)), ((), ())),
        preferred_element_type=jnp.float32) * SCALE
    row = jax.lax.broadcasted_iota(jnp.int32, (NCH, NB), 0)
    col = jax.lax.broadcasted_iota(jnp.int32, (NCH, NB), 1)
    g = jnp.where(col <= row, gating, NEG_INF)

    cols = []
    for _ in range(TOPK):
        m = jnp.max(g, axis=1, keepdims=True)
        idx = jnp.min(jnp.where(g == m, col, NB + 1), axis=1, keepdims=True)
        cols.append(idx)
        g = jnp.where(col == idx, -3.0e38, g)
    sel_ref[0, 0, :, :] = jnp.concatenate(cols, axis=1)


def _select(q4, k4, cos64, sin_sgn):
    slab = pl.BlockSpec((1, 1, S, DH), lambda b, h: (b, h, 0, 0))
    return pl.pallas_call(
        _select_body,
        grid=(B, H),
        in_specs=[
            slab, slab,
            pl.BlockSpec((S, DH), lambda b, h: (0, 0)),
            pl.BlockSpec((S, DH), lambda b, h: (0, 0)),
        ],
        out_specs=[
            slab, slab,
            pl.BlockSpec((1, 1, NCH, TOPK), lambda b, h: (b, h, 0, 0)),
        ],
        out_shape=[
            jax.ShapeDtypeStruct((B, H, S, DH), jnp.float32),
            jax.ShapeDtypeStruct((B, H, S, DH), jnp.float32),
            jax.ShapeDtypeStruct((B, H, NCH, TOPK), jnp.int32),
        ],
    )(q4, k4, cos64, sin_sgn)


def _attn_body(sel_ref, q_ref, k_ref, v_ref, o_ref):
    bh = pl.program_id(0)
    c = pl.program_id(1)
    base = (bh * NCH + c) * TOPK
    q = q_ref[0, 0, :, :]
    k_parts = []
    v_parts = []
    for i in range(TOPK):
        s = sel_ref[base + i]
        k_parts.append(k_ref[0, 0, pl.ds(s * BLOCK, BLOCK), :])
        v_parts.append(v_ref[0, 0, pl.ds(s * BLOCK, BLOCK), :])
    k_sel = jnp.concatenate(k_parts, axis=0)
    v_sel = jnp.concatenate(v_parts, axis=0)
    scores = jax.lax.dot_general(
        q, k_sel, (((1,), (1,)), ((), ())),
        preferred_element_type=jnp.float32) * SCALE
    m = jnp.max(scores, axis=1, keepdims=True)
    e = jnp.exp(scores - m)
    p = e / jnp.sum(e, axis=1, keepdims=True)
    o_ref[0, 0, :, :] = jnp.dot(p, v_sel, preferred_element_type=jnp.float32)


def _attention(qr, kr, v4, sel_flat):
    slab = pl.BlockSpec((1, 1, S, DH), lambda bh, c, *_: (bh // H, bh % H, 0, 0))
    grid_spec = pltpu.PrefetchScalarGridSpec(
        num_scalar_prefetch=1,
        grid=(B * H, NCH),
        in_specs=[
            pl.BlockSpec((1, 1, BLOCK, DH),
                         lambda bh, c, *_: (bh // H, bh % H, c, 0)),
            slab,
            slab,
        ],
        out_specs=pl.BlockSpec((1, 1, BLOCK, DH),
                               lambda bh, c, *_: (bh // H, bh % H, c, 0)),
    )
    return pl.pallas_call(
        _attn_body,
        grid_spec=grid_spec,
        out_shape=jax.ShapeDtypeStruct((B, H, S, DH), jnp.float32),
    )(sel_flat, qr, kr, v4)


def _oproj_body(a_ref, w_ref, o_ref):
    x_tile = jnp.concatenate([a_ref[0, h, :, :] for h in range(H)], axis=1)
    o_ref[0, :, :] = jnp.dot(x_tile, w_ref[...],
                             preferred_element_type=jnp.float32)


def _out_proj(attn, woT):
    return pl.pallas_call(
        _oproj_body,
        grid=(B, S // _TS),
        in_specs=[
            pl.BlockSpec((1, H, _TS, DH), lambda b, i: (b, 0, i, 0)),
            pl.BlockSpec((DIM, DIM), lambda b, i: (0, 0)),
        ],
        out_specs=pl.BlockSpec((1, _TS, DIM), lambda b, i: (b, i, 0)),
        out_shape=jax.ShapeDtypeStruct((B, S, DIM), jnp.float32),
    )(attn, woT)


@jax.jit
def _run(x, rope_cos, rope_sin, Wq, Wk, Wv, Wo):
    perm = jnp.asarray(_PERM)
    wqkv = jnp.concatenate([Wq[perm], Wk[perm], Wv], axis=0).T
    cos64 = jnp.concatenate([rope_cos, rope_cos], axis=1)
    sin_sgn = jnp.concatenate([-rope_sin, rope_sin], axis=1)

    q4, k4, v4 = _qkv_proj(x, wqkv)
    qr, kr, sel = _select(q4, k4, cos64, sin_sgn)
    attn = _attention(qr, kr, v4, sel.reshape(-1))
    return _out_proj(attn, Wo.T)


def kernel(x, rope_cos, rope_sin, Wq, Wk, Wv, Wo, layer_idx):
    return _run(x, rope_cos, rope_sin, Wq, Wk, Wv, Wo)


# speedup vs baseline: 972.5195x; 1.1289x over previous
import math

import jax
import jax.numpy as jnp
import numpy as np
from jax.experimental import pallas as pl
from jax.experimental.pallas import tpu as pltpu

B = 2
S = 2048
DIM = 1024
H = 16
DH = 64
BLOCK = 128
NB = S // BLOCK
NCH = S // BLOCK
TOPK = 8
NEG_INF = -10000.0
SCALE = 1.0 / math.sqrt(DH)

_TS = 512
_NT = S // _TS
_CPT = _TS // BLOCK


def _deinterleave_perm():
    perm = np.empty((DIM,), dtype=np.int32)
    for h in range(H):
        for j in range(DH):
            src = 2 * j if j < DH // 2 else 2 * (j - DH // 2) + 1
            perm[h * DH + j] = h * DH + src
    return perm


_PERM = _deinterleave_perm()


def _rope_full(t, cos_f, sin_f, half_mask):
    swapped = jnp.where(half_mask, jnp.roll(t, -DH // 2, axis=1),
                        jnp.roll(t, DH // 2, axis=1))
    return t * cos_f + swapped * sin_f


def _qkv_body(x_ref, w_ref, cos_ref, sin_ref,
              q_ref, k_ref, v_ref, qm_ref, br_ref):
    acc = jnp.dot(x_ref[0], w_ref[...], preferred_element_type=jnp.float32)
    lane = jax.lax.broadcasted_iota(jnp.int32, (_TS, DIM), 1)
    half_mask = (lane % DH) < (DH // 2)
    qr = _rope_full(acc[:, :DIM], cos_ref[...], sin_ref[...], half_mask)
    kr = _rope_full(acc[:, DIM:2 * DIM], cos_ref[...], sin_ref[...],
                    half_mask)
    for h in range(H):
        q_ref[0, h, :, :] = qr[:, h * DH:(h + 1) * DH].astype(jnp.bfloat16)
        k_ref[0, h, :, :] = kr[:, h * DH:(h + 1) * DH].astype(jnp.bfloat16)
        v_ref[0, h, :, :] = acc[:, 2 * DIM + h * DH:
                                2 * DIM + (h + 1) * DH].astype(jnp.bfloat16)
    qm_ref[0, 0] = jnp.concatenate(
        [qr[c * BLOCK + BLOCK // 2:c * BLOCK + BLOCK // 2 + 1, :]
         for c in range(_CPT)], axis=0)
    br_ref[0, 0] = jnp.mean(kr.reshape(_CPT, BLOCK, DIM), axis=1)


def _qkv_proj(x3d, wqkv, cos_f, sin_f):
    obf = jax.ShapeDtypeStruct((B, H, S, DH), jnp.bfloat16)
    ospec = pl.BlockSpec((1, H, _TS, DH), lambda b, i: (b, 0, i, 0))
    small = pl.BlockSpec((1, 1, _CPT, DIM), lambda b, i: (b, i, 0, 0))
    return pl.pallas_call(
        _qkv_body,
        grid=(B, _NT),
        in_specs=[
            pl.BlockSpec((1, _TS, DIM), lambda b, i: (b, i, 0)),
            pl.BlockSpec((DIM, 3 * DIM), lambda b, i: (0, 0)),
            pl.BlockSpec((_TS, DIM), lambda b, i: (i, 0)),
            pl.BlockSpec((_TS, DIM), lambda b, i: (i, 0)),
        ],
        out_specs=[ospec, ospec, ospec, small, small],
        out_shape=[obf, obf, obf,
                   jax.ShapeDtypeStruct((B, _NT, _CPT, DIM), jnp.float32),
                   jax.ShapeDtypeStruct((B, _NT, _CPT, DIM), jnp.float32)],
    )(x3d, wqkv, cos_f, sin_f)


def _select_body(qm_ref, br_ref, sel_ref):
    qm = qm_ref[0]
    br = br_ref[0]
    gs = []
    for h in range(H):
        g_h = jax.lax.dot_general(
            qm[:, h * DH:(h + 1) * DH], br[:, h * DH:(h + 1) * DH],
            (((1,), (1,)), ((), ())),
            preferred_element_type=jnp.float32) * SCALE
        gs.append(g_h)
    g = jnp.concatenate(gs, axis=0)
    rows = H * NCH
    row = jax.lax.broadcasted_iota(jnp.int32, (rows, NB), 0) % NCH
    col = jax.lax.broadcasted_iota(jnp.int32, (rows, NB), 1)
    g = jnp.where(col <= row, g, NEG_INF)

    cols = []
    for _ in range(TOPK):
        m = jnp.max(g, axis=1, keepdims=True)
        idx = jnp.min(jnp.where(g == m, col, NB + 1), axis=1, keepdims=True)
        cols.append(idx)
        g = jnp.where(col == idx, -3.0e38, g)
    sel_ref[0] = jnp.concatenate(cols, axis=1)


def _select(qmid, brep):
    return pl.pallas_call(
        _select_body,
        grid=(B,),
        in_specs=[
            pl.BlockSpec((1, NCH, DIM), lambda b: (b, 0, 0)),
            pl.BlockSpec((1, NB, DIM), lambda b: (b, 0, 0)),
        ],
        out_specs=pl.BlockSpec((1, H * NCH, TOPK), lambda b: (b, 0, 0)),
        out_shape=jax.ShapeDtypeStruct((B, H * NCH, TOPK), jnp.int32),
    )(qmid, brep)


def _attn_body(sel_ref, q_ref, k_ref, v_ref, o_ref):
    bh = pl.program_id(0)
    c = pl.program_id(1)
    base = (bh * NCH + c) * TOPK
    q = q_ref[0, 0, :, :]
    k_parts = []
    v_parts = []
    for i in range(TOPK):
        s = sel_ref[base + i]
        k_parts.append(k_ref[0, 0, pl.ds(s * BLOCK, BLOCK), :])
        v_parts.append(v_ref[0, 0, pl.ds(s * BLOCK, BLOCK), :])
    k_sel = jnp.concatenate(k_parts, axis=0)
    v_sel = jnp.concatenate(v_parts, axis=0)
    scores = jax.lax.dot_general(
        q, k_sel, (((1,), (1,)), ((), ())),
        preferred_element_type=jnp.float32) * SCALE
    m = jnp.max(scores, axis=1, keepdims=True)
    e = jnp.exp(scores - m)
    p = (e / jnp.sum(e, axis=1, keepdims=True)).astype(jnp.bfloat16)
    o_ref[0, 0, :, :] = jnp.dot(
        p, v_sel, preferred_element_type=jnp.float32).astype(jnp.bfloat16)


def _attention(qbf, kbf, vbf, sel_flat):
    slab = pl.BlockSpec((1, 1, S, DH),
                        lambda bh, c, *_: (bh // H, bh % H, 0, 0))
    grid_spec = pltpu.PrefetchScalarGridSpec(
        num_scalar_prefetch=1,
        grid=(B * H, NCH),
        in_specs=[
            pl.BlockSpec((1, 1, BLOCK, DH),
                         lambda bh, c, *_: (bh // H, bh % H, c, 0)),
            slab,
            slab,
        ],
        out_specs=pl.BlockSpec((1, 1, BLOCK, DH),
                               lambda bh, c, *_: (bh // H, bh % H, c, 0)),
    )
    return pl.pallas_call(
        _attn_body,
        grid_spec=grid_spec,
        out_shape=jax.ShapeDtypeStruct((B, H, S, DH), jnp.bfloat16),
    )(sel_flat, qbf, kbf, vbf)


def _oproj_body(a_ref, w_ref, o_ref):
    x_tile = jnp.concatenate([a_ref[0, h, :, :] for h in range(H)], axis=1)
    o_ref[0, :, :] = jnp.dot(x_tile, w_ref[...],
                             preferred_element_type=jnp.float32)


def _out_proj(attn, woT):
    return pl.pallas_call(
        _oproj_body,
        grid=(B, _NT),
        in_specs=[
            pl.BlockSpec((1, H, _TS, DH), lambda b, i: (b, 0, i, 0)),
            pl.BlockSpec((DIM, DIM), lambda b, i: (0, 0)),
        ],
        out_specs=pl.BlockSpec((1, _TS, DIM), lambda b, i: (b, i, 0)),
        out_shape=jax.ShapeDtypeStruct((B, S, DIM), jnp.float32),
    )(attn, woT)


@jax.jit
def _run(x, rope_cos, rope_sin, Wq, Wk, Wv, Wo):
    perm = jnp.asarray(_PERM)
    wqkv = jnp.concatenate([Wq[perm], Wk[perm], Wv], axis=0).T
    cos64 = jnp.concatenate([rope_cos, rope_cos], axis=1)
    sin_sgn = jnp.concatenate([-rope_sin, rope_sin], axis=1)
    cos_f = jnp.tile(cos64, (1, H))
    sin_f = jnp.tile(sin_sgn, (1, H))

    qbf, kbf, vbf, qm4, br4 = _qkv_proj(x, wqkv, cos_f, sin_f)
    sel = _select(qm4.reshape(B, NCH, DIM), br4.reshape(B, NB, DIM))
    attn = _attention(qbf, kbf, vbf, sel.reshape(-1))
    return _out_proj(attn, Wo.T.astype(jnp.bfloat16))


def kernel(x, rope_cos, rope_sin, Wq, Wk, Wv, Wo, layer_idx):
    return _run(x, rope_cos, rope_sin, Wq, Wk, Wv, Wo)


# speedup vs baseline: 1346.6046x; 1.3847x over previous
import math

import jax
import jax.numpy as jnp
import numpy as np
from jax.experimental import pallas as pl
from jax.experimental.pallas import tpu as pltpu

B = 2
S = 2048
DIM = 1024
H = 16
DH = 64
BLOCK = 128
NB = S // BLOCK
NCH = S // BLOCK
TOPK = 8
NEG_INF = -10000.0
SCALE = 1.0 / math.sqrt(DH)

_TS = 512
_NT = S // _TS
_CPT = _TS // BLOCK


def _deinterleave_perm():
    perm = np.empty((DIM,), dtype=np.int32)
    for h in range(H):
        for j in range(DH):
            src = 2 * j if j < DH // 2 else 2 * (j - DH // 2) + 1
            perm[h * DH + j] = h * DH + src
    return perm


_PERM = _deinterleave_perm()


def _rope_full(t, cos_f, sin_f, half_mask):
    swapped = jnp.where(half_mask, jnp.roll(t, -DH // 2, axis=1),
                        jnp.roll(t, DH // 2, axis=1))
    return t * cos_f + swapped * sin_f


def _qkv_body(x_ref, w_ref, cos_ref, sin_ref,
              q_ref, k_ref, v_ref, qm_ref, br_ref):
    acc = jnp.dot(x_ref[0], w_ref[...], preferred_element_type=jnp.float32)
    lane = jax.lax.broadcasted_iota(jnp.int32, (_TS, DIM), 1)
    half_mask = (lane % DH) < (DH // 2)
    qr = _rope_full(acc[:, :DIM], cos_ref[...], sin_ref[...], half_mask)
    kr = _rope_full(acc[:, DIM:2 * DIM], cos_ref[...], sin_ref[...],
                    half_mask)
    for h in range(H):
        q_ref[0, h, :, :] = qr[:, h * DH:(h + 1) * DH].astype(jnp.bfloat16)
        k_ref[0, h, :, :] = kr[:, h * DH:(h + 1) * DH].astype(jnp.bfloat16)
        v_ref[0, h, :, :] = acc[:, 2 * DIM + h * DH:
                                2 * DIM + (h + 1) * DH].astype(jnp.bfloat16)
    qm_ref[0, 0] = jnp.concatenate(
        [qr[c * BLOCK + BLOCK // 2:c * BLOCK + BLOCK // 2 + 1, :]
         for c in range(_CPT)], axis=0)
    br_ref[0, 0] = jnp.mean(kr.reshape(_CPT, BLOCK, DIM), axis=1)


def _qkv_proj(x3d, wqkv, cos_f, sin_f):
    obf = jax.ShapeDtypeStruct((B, H, S, DH), jnp.bfloat16)
    ospec = pl.BlockSpec((1, H, _TS, DH), lambda b, i: (b, 0, i, 0))
    small = pl.BlockSpec((1, 1, _CPT, DIM), lambda b, i: (b, i, 0, 0))
    return pl.pallas_call(
        _qkv_body,
        grid=(B, _NT),
        in_specs=[
            pl.BlockSpec((1, _TS, DIM), lambda b, i: (b, i, 0)),
            pl.BlockSpec((DIM, 3 * DIM), lambda b, i: (0, 0)),
            pl.BlockSpec((_TS, DIM), lambda b, i: (i, 0)),
            pl.BlockSpec((_TS, DIM), lambda b, i: (i, 0)),
        ],
        out_specs=[ospec, ospec, ospec, small, small],
        out_shape=[obf, obf, obf,
                   jax.ShapeDtypeStruct((B, _NT, _CPT, DIM), jnp.float32),
                   jax.ShapeDtypeStruct((B, _NT, _CPT, DIM), jnp.float32)],
    )(x3d, wqkv, cos_f, sin_f)


def _select_body(qm_ref, br_ref, sel_ref):
    qm = qm_ref[0]
    br = br_ref[0]
    gs = []
    for h in range(H):
        g_h = jax.lax.dot_general(
            qm[:, h * DH:(h + 1) * DH], br[:, h * DH:(h + 1) * DH],
            (((1,), (1,)), ((), ())),
            preferred_element_type=jnp.float32) * SCALE
        gs.append(g_h)
    g = jnp.concatenate(gs, axis=0)
    rows = H * NCH
    row = jax.lax.broadcasted_iota(jnp.int32, (rows, NB), 0) % NCH
    col = jax.lax.broadcasted_iota(jnp.int32, (rows, NB), 1)
    g = jnp.where(col <= row, g, NEG_INF)

    cols = []
    for _ in range(TOPK):
        m = jnp.max(g, axis=1, keepdims=True)
        idx = jnp.min(jnp.where(g == m, col, NB + 1), axis=1, keepdims=True)
        cols.append(idx)
        g = jnp.where(col == idx, -3.0e38, g)
    sel_ref[0] = jnp.concatenate(cols, axis=1)


def _select(qmid, brep):
    return pl.pallas_call(
        _select_body,
        grid=(B,),
        in_specs=[
            pl.BlockSpec((1, NCH, DIM), lambda b: (b, 0, 0)),
            pl.BlockSpec((1, NB, DIM), lambda b: (b, 0, 0)),
        ],
        out_specs=pl.BlockSpec((1, H * NCH, TOPK), lambda b: (b, 0, 0)),
        out_shape=jax.ShapeDtypeStruct((B, H * NCH, TOPK), jnp.int32),
    )(qmid, brep)


def _attn_body(sel_ref, q_ref, k_ref, v_ref, o_ref):
    bh = pl.program_id(0)
    for c in range(NCH):
        base = (bh * NCH + c) * TOPK
        q = q_ref[0, 0, c * BLOCK:(c + 1) * BLOCK, :]
        k_parts = []
        v_parts = []
        for i in range(TOPK):
            s = sel_ref[base + i]
            k_parts.append(k_ref[0, 0, pl.ds(s * BLOCK, BLOCK), :])
            v_parts.append(v_ref[0, 0, pl.ds(s * BLOCK, BLOCK), :])
        k_sel = jnp.concatenate(k_parts, axis=0)
        v_sel = jnp.concatenate(v_parts, axis=0)
        scores = jax.lax.dot_general(
            q, k_sel, (((1,), (1,)), ((), ())),
            preferred_element_type=jnp.float32) * SCALE
        m = jnp.max(scores, axis=1, keepdims=True)
        e = jnp.exp(scores - m)
        p = (e / jnp.sum(e, axis=1, keepdims=True)).astype(jnp.bfloat16)
        o_ref[0, 0, c * BLOCK:(c + 1) * BLOCK, :] = jnp.dot(
            p, v_sel, preferred_element_type=jnp.float32).astype(jnp.bfloat16)


def _attention(qbf, kbf, vbf, sel_flat):
    slab = pl.BlockSpec((1, 1, S, DH),
                        lambda bh, *_: (bh // H, bh % H, 0, 0))
    grid_spec = pltpu.PrefetchScalarGridSpec(
        num_scalar_prefetch=1,
        grid=(B * H,),
        in_specs=[slab, slab, slab],
        out_specs=slab,
    )
    return pl.pallas_call(
        _attn_body,
        grid_spec=grid_spec,
        out_shape=jax.ShapeDtypeStruct((B, H, S, DH), jnp.bfloat16),
    )(sel_flat, qbf, kbf, vbf)


def _oproj_body(a_ref, w_ref, o_ref):
    x_tile = jnp.concatenate([a_ref[0, h, :, :] for h in range(H)], axis=1)
    o_ref[0, :, :] = jnp.dot(x_tile, w_ref[...],
                             preferred_element_type=jnp.float32)


def _out_proj(attn, woT):
    return pl.pallas_call(
        _oproj_body,
        grid=(B, _NT),
        in_specs=[
            pl.BlockSpec((1, H, _TS, DH), lambda b, i: (b, 0, i, 0)),
            pl.BlockSpec((DIM, DIM), lambda b, i: (0, 0)),
        ],
        out_specs=pl.BlockSpec((1, _TS, DIM), lambda b, i: (b, i, 0)),
        out_shape=jax.ShapeDtypeStruct((B, S, DIM), jnp.float32),
    )(attn, woT)


@jax.jit
def _run(x, rope_cos, rope_sin, Wq, Wk, Wv, Wo):
    perm = jnp.asarray(_PERM)
    wqkv = jnp.concatenate([Wq[perm], Wk[perm], Wv], axis=0).T
    cos64 = jnp.concatenate([rope_cos, rope_cos], axis=1)
    sin_sgn = jnp.concatenate([-rope_sin, rope_sin], axis=1)
    cos_f = jnp.tile(cos64, (1, H))
    sin_f = jnp.tile(sin_sgn, (1, H))

    qbf, kbf, vbf, qm4, br4 = _qkv_proj(x, wqkv, cos_f, sin_f)
    sel = _select(qm4.reshape(B, NCH, DIM), br4.reshape(B, NB, DIM))
    attn = _attention(qbf, kbf, vbf, sel.reshape(-1))
    return _out_proj(attn, Wo.T.astype(jnp.bfloat16))


def kernel(x, rope_cos, rope_sin, Wq, Wk, Wv, Wo, layer_idx):
    return _run(x, rope_cos, rope_sin, Wq, Wk, Wv, Wo)


# speedup vs baseline: 1861.1932x; 1.3821x over previous
import math

import jax
import jax.numpy as jnp
import numpy as np
from jax.experimental import pallas as pl
from jax.experimental.pallas import tpu as pltpu

B = 2
S = 2048
DIM = 1024
H = 16
DH = 64
BLOCK = 128
NB = S // BLOCK
NCH = S // BLOCK
TOPK = 8
NEG_INF = -10000.0
SCALE = 1.0 / math.sqrt(DH)

_TS = 512
_NT = S // _TS
_CPT = _TS // BLOCK


def _deinterleave_perm():
    perm = np.empty((DIM,), dtype=np.int32)
    for h in range(H):
        for j in range(DH):
            src = 2 * j if j < DH // 2 else 2 * (j - DH // 2) + 1
            perm[h * DH + j] = h * DH + src
    return perm


_PERM = _deinterleave_perm()


def _rope_full(t, cos_f, sin_f):
    lane = jax.lax.broadcasted_iota(jnp.int32, t.shape, 1)
    half_mask = (lane % DH) < (DH // 2)
    swapped = jnp.where(half_mask, jnp.roll(t, -DH // 2, axis=1),
                        jnp.roll(t, DH // 2, axis=1))
    return t * cos_f + swapped * sin_f


def _qkv_body(x_ref, wk_ref, wqv_ref, cos_ref, sin_ref,
              q_ref, k_ref, v_ref, br_ref):
    x = x_ref[0]
    kr = _rope_full(
        jnp.dot(x, wk_ref[...], preferred_element_type=jnp.float32),
        cos_ref[...], sin_ref[...])
    acc = jnp.dot(x.astype(jnp.bfloat16), wqv_ref[...],
                  preferred_element_type=jnp.float32)
    qr = _rope_full(acc[:, :DIM], cos_ref[...], sin_ref[...])
    for h in range(H):
        q_ref[0, h, :, :] = qr[:, h * DH:(h + 1) * DH].astype(jnp.bfloat16)
        k_ref[0, h, :, :] = kr[:, h * DH:(h + 1) * DH].astype(jnp.bfloat16)
        v_ref[0, h, :, :] = acc[:, DIM + h * DH:
                                DIM + (h + 1) * DH].astype(jnp.bfloat16)
    br_ref[0, 0] = jnp.mean(kr.reshape(_CPT, BLOCK, DIM), axis=1)


def _qkv_proj(x3d, wk, wqv, cos_f, sin_f):
    obf = jax.ShapeDtypeStruct((B, H, S, DH), jnp.bfloat16)
    ospec = pl.BlockSpec((1, H, _TS, DH), lambda b, i: (b, 0, i, 0))
    return pl.pallas_call(
        _qkv_body,
        grid=(B, _NT),
        in_specs=[
            pl.BlockSpec((1, _TS, DIM), lambda b, i: (b, i, 0)),
            pl.BlockSpec((DIM, DIM), lambda b, i: (0, 0)),
            pl.BlockSpec((DIM, 2 * DIM), lambda b, i: (0, 0)),
            pl.BlockSpec((_TS, DIM), lambda b, i: (i, 0)),
            pl.BlockSpec((_TS, DIM), lambda b, i: (i, 0)),
        ],
        out_specs=[ospec, ospec, ospec,
                   pl.BlockSpec((1, 1, _CPT, DIM), lambda b, i: (b, i, 0, 0))],
        out_shape=[obf, obf, obf,
                   jax.ShapeDtypeStruct((B, _NT, _CPT, DIM), jnp.float32)],
    )(x3d, wk, wqv, cos_f, sin_f)


def _select_body(xm_ref, wq_ref, cosm_ref, sinm_ref, br_ref, sel_ref):
    qm = _rope_full(
        jnp.dot(xm_ref[0], wq_ref[...], preferred_element_type=jnp.float32),
        cosm_ref[...], sinm_ref[...])
    br = br_ref[0]
    gs = []
    for h in range(H):
        g_h = jax.lax.dot_general(
            qm[:, h * DH:(h + 1) * DH], br[:, h * DH:(h + 1) * DH],
            (((1,), (1,)), ((), ())),
            preferred_element_type=jnp.float32) * SCALE
        gs.append(g_h)
    g = jnp.concatenate(gs, axis=0)
    rows = H * NCH
    row = jax.lax.broadcasted_iota(jnp.int32, (rows, NB), 0) % NCH
    col = jax.lax.broadcasted_iota(jnp.int32, (rows, NB), 1)
    g = jnp.where(col <= row, g, NEG_INF)

    cols = []
    for _ in range(TOPK):
        m = jnp.max(g, axis=1, keepdims=True)
        idx = jnp.min(jnp.where(g == m, col, NB + 1), axis=1, keepdims=True)
        cols.append(idx)
        g = jnp.where(col == idx, -3.0e38, g)
    sel_ref[0] = jnp.concatenate(cols, axis=1)


def _select(x_mid, wq, cos_mid, sin_mid, brep):
    return pl.pallas_call(
        _select_body,
        grid=(B,),
        in_specs=[
            pl.BlockSpec((1, NCH, DIM), lambda b: (b, 0, 0)),
            pl.BlockSpec((DIM, DIM), lambda b: (0, 0)),
            pl.BlockSpec((NCH, DIM), lambda b: (0, 0)),
            pl.BlockSpec((NCH, DIM), lambda b: (0, 0)),
            pl.BlockSpec((1, NB, DIM), lambda b: (b, 0, 0)),
        ],
        out_specs=pl.BlockSpec((1, H * NCH, TOPK), lambda b: (b, 0, 0)),
        out_shape=jax.ShapeDtypeStruct((B, H * NCH, TOPK), jnp.int32),
    )(x_mid, wq, cos_mid, sin_mid, brep)


def _attn_body(sel_ref, q_ref, k_ref, v_ref, o_ref):
    bh = pl.program_id(0)
    for c in range(NCH):
        base = (bh * NCH + c) * TOPK
        q = q_ref[0, 0, c * BLOCK:(c + 1) * BLOCK, :]
        k_parts = []
        v_parts = []
        for i in range(TOPK):
            s = sel_ref[base + i]
            k_parts.append(k_ref[0, 0, pl.ds(s * BLOCK, BLOCK), :])
            v_parts.append(v_ref[0, 0, pl.ds(s * BLOCK, BLOCK), :])
        k_sel = jnp.concatenate(k_parts, axis=0)
        v_sel = jnp.concatenate(v_parts, axis=0)
        scores = jax.lax.dot_general(
            q, k_sel, (((1,), (1,)), ((), ())),
            preferred_element_type=jnp.float32) * SCALE
        e = jnp.exp(scores)
        denom = jnp.sum(e, axis=1, keepdims=True)
        pv = jnp.dot(e.astype(jnp.bfloat16), v_sel,
                     preferred_element_type=jnp.float32)
        o_ref[0, 0, c * BLOCK:(c + 1) * BLOCK, :] = (
            pv / denom).astype(jnp.bfloat16)


def _attention(qbf, kbf, vbf, sel_flat):
    slab = pl.BlockSpec((1, 1, S, DH),
                        lambda bh, *_: (bh // H, bh % H, 0, 0))
    grid_spec = pltpu.PrefetchScalarGridSpec(
        num_scalar_prefetch=1,
        grid=(B * H,),
        in_specs=[slab, slab, slab],
        out_specs=slab,
    )
    return pl.pallas_call(
        _attn_body,
        grid_spec=grid_spec,
        out_shape=jax.ShapeDtypeStruct((B, H, S, DH), jnp.bfloat16),
    )(sel_flat, qbf, kbf, vbf)


def _oproj_body(a_ref, w_ref, o_ref):
    x_tile = jnp.concatenate([a_ref[0, h, :, :] for h in range(H)], axis=1)
    o_ref[0, :, :] = jnp.dot(x_tile, w_ref[...],
                             preferred_element_type=jnp.float32)


def _out_proj(attn, woT):
    return pl.pallas_call(
        _oproj_body,
        grid=(B, _NT),
        in_specs=[
            pl.BlockSpec((1, H, _TS, DH), lambda b, i: (b, 0, i, 0)),
            pl.BlockSpec((DIM, DIM), lambda b, i: (0, 0)),
        ],
        out_specs=pl.BlockSpec((1, _TS, DIM), lambda b, i: (b, i, 0)),
        out_shape=jax.ShapeDtypeStruct((B, S, DIM), jnp.float32),
    )(attn, woT)


@jax.jit
def _run(x, rope_cos, rope_sin, Wq, Wk, Wv, Wo):
    perm = jnp.asarray(_PERM)
    wk = Wk[perm].T
    wqv = jnp.concatenate([Wq[perm], Wv], axis=0).T.astype(jnp.bfloat16)
    cos64 = jnp.concatenate([rope_cos, rope_cos], axis=1)
    sin_sgn = jnp.concatenate([-rope_sin, rope_sin], axis=1)
    cos_f = jnp.tile(cos64, (1, H))
    sin_f = jnp.tile(sin_sgn, (1, H))

    qbf, kbf, vbf, br4 = _qkv_proj(x, wk, wqv, cos_f, sin_f)
    x_mid = x[:, BLOCK // 2::BLOCK, :]
    sel = _select(x_mid, Wq[perm].T, cos_f[BLOCK // 2::BLOCK],
                  sin_f[BLOCK // 2::BLOCK], br4.reshape(B, NB, DIM))
    attn = _attention(qbf, kbf, vbf, sel.reshape(-1))
    return _out_proj(attn, Wo.T.astype(jnp.bfloat16))


def kernel(x, rope_cos, rope_sin, Wq, Wk, Wv, Wo, layer_idx):
    return _run(x, rope_cos, rope_sin, Wq, Wk, Wv, Wo)
